# baseline (speedup 1.0000x reference)
import jax
import jax.numpy as jnp
from jax.experimental import pallas as pl
from jax.experimental.pallas import tpu as pltpu

N = 4096
F = 128
H = 32
OUT = 64
BLK = 1024
P = N // BLK
PA = P * P
PB = 4
TOTAL = PA + PB


def _leaky(x):
    return jnp.where(x > 0, x, 0.01 * x)


def _tile_pq(t):
    qa = t // P
    ia = t % P
    pa = jnp.where(ia == P - 1, qa, ia + (ia >= qa).astype(jnp.int32))
    u = t - PA
    pb = jnp.where(u == 0, 2, 3)
    qb = jnp.where(u == 0, 1, u - 1)
    p = jnp.where(t < PA, pa, pb)
    q = jnp.where(t < PA, qa, qb)
    return p, q


def _pre_body(x0, x1, x2, w1s, y_ref):
    for b, xr in enumerate((x0, x1, x2)):
        y_ref[:, b * H:(b + 1) * H] = jnp.dot(
            _leaky(xr[...]), w1s[b], preferred_element_type=jnp.float32)


def _main_body(l0, l1, l2, y, b1s, w2s, b2s, w3s, b3s,
               o_ref, c_sc, v_sc, z_sc, h_sc):
    t = pl.program_id(0)
    p, q = _tile_pq(t)
    ia = t % P
    phase_a = t < PA
    v_exec = jnp.logical_or(jnp.logical_not(phase_a),
                            jnp.logical_or(p < q, ia == P - 1))

    @pl.when(t == 0)
    def _():
        c_sc[...] = jnp.zeros_like(c_sc)
        v_sc[...] = jnp.zeros_like(v_sc)
        z_sc[...] = jnp.zeros_like(z_sc)

    for b, lr in enumerate((l0, l1, l2)):
        lb = lr[...]

        csum = jnp.sum(lb, axis=0, keepdims=True)

        @pl.when(phase_a)
        def _(b=b, lb=lb, csum=csum):
            c_sc[b, :, pl.ds(q * BLK, BLK)] += csum
            zc = jnp.dot(lb, y[:, b * H:(b + 1) * H],
                         preferred_element_type=jnp.float32)
            z_sc[pl.ds(p * BLK, BLK), b * H:(b + 1) * H] += zc

        @pl.when(t == 0)
        def _(b=b, lb=lb):
            h_sc[b, 0] = lb

        @pl.when(t == 1)
        def _(b=b, lb=lb):
            h_sc[b, 1] = lb

        @pl.when(v_exec)
        def _(b=b, lb=lb):
            cp_col = c_sc[b, :, pl.ds(p * BLK, BLK)].reshape(BLK, 1)
            v_sc[b, :, pl.ds(q * BLK, BLK)] += jnp.sum(
                lb * cp_col, axis=0, keepdims=True)

        @pl.when(t == PA)
        def _(b=b):
            w = c_sc[b, :, pl.ds(1 * BLK, BLK)].reshape(BLK, 1)
            v_sc[b, :, pl.ds(0, BLK)] += jnp.sum(
                h_sc[b, 0] * w, axis=0, keepdims=True)

        @pl.when(t == PA + 1)
        def _(b=b):
            w = c_sc[b, :, pl.ds(2 * BLK, BLK)].reshape(BLK, 1)
            v_sc[b, :, pl.ds(0, BLK)] += jnp.sum(
                h_sc[b, 1] * w, axis=0, keepdims=True)

    @pl.when(t == TOTAL - 1)
    def _():
        s = jnp.zeros((1, OUT), jnp.float32)
        for b in range(3):
            a = _leaky(z_sc[:, b * H:(b + 1) * H] + b1s[b][None, :])
            tb = jnp.dot(v_sc[b], a, preferred_element_type=jnp.float32)
            u = jnp.dot(tb, w2s[b], preferred_element_type=jnp.float32)
            u = u + jnp.sum(c_sc[b]) * b2s[b][None, :]
            s = s + jnp.dot(u, w3s[b], preferred_element_type=jnp.float32)
            s = s + jnp.float32(N) * b3s[b][None, :]
        m = jnp.max(s)
        e = jnp.exp(s - m)
        o_ref[...] = e / jnp.sum(e)


def kernel(X0, X1, X2, L0, L1, L2, batch0, batch1, batch2,
           W01, b01, W02, b02, W03, b03,
           W11, b11, W12, b12, W13, b13,
           W21, b21, W22, b22, W23, b23):
    w1s = jnp.stack([W01, W11, W21])
    b1s = jnp.stack([b01, b11, b21])
    w2s = jnp.stack([W02, W12, W22])
    b2s = jnp.stack([b02, b12, b22])
    w3s = jnp.stack([W03, W13, W23])
    b3s = jnp.stack([b03, b13, b23])

    y1 = pl.pallas_call(
        _pre_body,
        out_shape=jax.ShapeDtypeStruct((N, 3 * H), jnp.float32),
    )(X0, X1, X2, w1s)

    lspec = pl.BlockSpec((BLK, BLK), _tile_pq,
                         pipeline_mode=pl.Buffered(buffer_count=2))
    const = lambda shape: pl.BlockSpec(shape, lambda t: (0,) * len(shape))

    def _ymap(t):
        _, q = _tile_pq(t)
        return jnp.where(t < PA, q, P - 1), 0

    out = pl.pallas_call(
        _main_body,
        grid=(TOTAL,),
        in_specs=[lspec, lspec, lspec,
                  pl.BlockSpec((BLK, 3 * H), _ymap),
                  const((3, H)),
                  const((3, H, H)),
                  const((3, H)),
                  const((3, H, OUT)),
                  const((3, OUT))],
        out_specs=pl.BlockSpec((1, OUT), lambda t: (0, 0)),
        out_shape=jax.ShapeDtypeStruct((1, OUT), jnp.float32),
        scratch_shapes=[pltpu.VMEM((3, 1, N), jnp.float32),
                        pltpu.VMEM((3, 1, N), jnp.float32),
                        pltpu.VMEM((N, 3 * H), jnp.float32),
                        pltpu.VMEM((3, 2, BLK, BLK), jnp.float32)],
    )(L0, L1, L2, y1, b1s, w2s, b2s, w3s, b3s)
    return out


# speedup vs baseline: 1.0093x; 1.0093x over previous
import jax
import jax.numpy as jnp
from jax.experimental import pallas as pl
from jax.experimental.pallas import tpu as pltpu

N = 4096
F = 128
H = 32
OUT = 64
BLK = 1024
P = N // BLK
PA = P * P
PB = 4
TOTAL = PA + PB


def _leaky(x):
    return jnp.where(x > 0, x, 0.01 * x)


def _tile_pq(t):
    qa = t // P
    ia = t % P
    pa = jnp.where(ia == P - 1, qa, ia + (ia >= qa).astype(jnp.int32))
    u = t - PA
    pb = jnp.where(u == 0, 2, 3)
    qb = jnp.where(u == 0, 1, u - 1)
    p = jnp.where(t < PA, pa, pb)
    q = jnp.where(t < PA, qa, qb)
    return p, q


def _pre_body(x0, x1, x2, w1s, y_ref):
    for b, xr in enumerate((x0, x1, x2)):
        y_ref[:, b * H:(b + 1) * H] = jnp.dot(
            _leaky(xr[...]), w1s[b], preferred_element_type=jnp.float32)


def _main_body(l0, l1, l2, y, b1s, w2s, b2s, w3s, b3s,
               o_ref, c_sc, v_sc, z_sc, h_sc, tb_sc):
    t = pl.program_id(0)
    p, q = _tile_pq(t)
    ia = t % P
    phase_a = t < PA
    v_exec = jnp.logical_or(jnp.logical_not(phase_a),
                            jnp.logical_or(p < q, ia == P - 1))

    @pl.when(t == 0)
    def _():
        c_sc[...] = jnp.zeros_like(c_sc)
        v_sc[...] = jnp.zeros_like(v_sc)
        z_sc[...] = jnp.zeros_like(z_sc)

    for b, lr in enumerate((l0, l1, l2)):
        lb = lr[...]

        csum = jnp.sum(lb, axis=0, keepdims=True)

        @pl.when(phase_a)
        def _(b=b, lb=lb, csum=csum):
            c_sc[b, :, pl.ds(q * BLK, BLK)] += csum
            zc = jnp.dot(lb, y[:, b * H:(b + 1) * H],
                         preferred_element_type=jnp.float32)
            z_sc[pl.ds(p * BLK, BLK), b * H:(b + 1) * H] += zc

        @pl.when(t == 0)
        def _(b=b, lb=lb):
            h_sc[b, 0] = lb

        @pl.when(t == 1)
        def _(b=b, lb=lb):
            h_sc[b, 1] = lb

        @pl.when(v_exec)
        def _(b=b, lb=lb):
            cp_col = c_sc[b, :, pl.ds(p * BLK, BLK)].reshape(BLK, 1)
            v_sc[b, :, pl.ds(q * BLK, BLK)] += jnp.sum(
                lb * cp_col, axis=0, keepdims=True)

        @pl.when(t == PA)
        def _(b=b):
            w = c_sc[b, :, pl.ds(1 * BLK, BLK)].reshape(BLK, 1)
            v_sc[b, :, pl.ds(0, BLK)] += jnp.sum(
                h_sc[b, 0] * w, axis=0, keepdims=True)

        @pl.when(t == PA + 1)
        def _(b=b):
            w = c_sc[b, :, pl.ds(2 * BLK, BLK)].reshape(BLK, 1)
            v_sc[b, :, pl.ds(0, BLK)] += jnp.sum(
                h_sc[b, 1] * w, axis=0, keepdims=True)

    def _tb_seg(seg, init):
        for b in range(3):
            sl = pl.ds(seg * BLK, BLK)
            a = _leaky(z_sc[sl, b * H:(b + 1) * H] + b1s[b][None, :])
            part = jnp.dot(v_sc[b, :, sl], a,
                           preferred_element_type=jnp.float32)
            if init:
                tb_sc[b] = part
            else:
                tb_sc[b] += part

    @pl.when(t == PA)
    def _():
        _tb_seg(3, True)

    @pl.when(t == PA + 2)
    def _():
        _tb_seg(0, False)

    @pl.when(t == TOTAL - 1)
    def _():
        _tb_seg(1, False)
        _tb_seg(2, False)
        s = jnp.zeros((1, OUT), jnp.float32)
        for b in range(3):
            u = jnp.dot(tb_sc[b], w2s[b], preferred_element_type=jnp.float32)
            u = u + jnp.sum(c_sc[b]) * b2s[b][None, :]
            s = s + jnp.dot(u, w3s[b], preferred_element_type=jnp.float32)
            s = s + jnp.float32(N) * b3s[b][None, :]
        m = jnp.max(s)
        e = jnp.exp(s - m)
        o_ref[...] = e / jnp.sum(e)


def kernel(X0, X1, X2, L0, L1, L2, batch0, batch1, batch2,
           W01, b01, W02, b02, W03, b03,
           W11, b11, W12, b12, W13, b13,
           W21, b21, W22, b22, W23, b23):
    w1s = jnp.stack([W01, W11, W21])
    b1s = jnp.stack([b01, b11, b21])
    w2s = jnp.stack([W02, W12, W22])
    b2s = jnp.stack([b02, b12, b22])
    w3s = jnp.stack([W03, W13, W23])
    b3s = jnp.stack([b03, b13, b23])

    y1 = pl.pallas_call(
        _pre_body,
        out_shape=jax.ShapeDtypeStruct((N, 3 * H), jnp.float32),
    )(X0, X1, X2, w1s)

    lspec = pl.BlockSpec((BLK, BLK), _tile_pq,
                         pipeline_mode=pl.Buffered(buffer_count=2))
    const = lambda shape: pl.BlockSpec(shape, lambda t: (0,) * len(shape))

    def _ymap(t):
        _, q = _tile_pq(t)
        return jnp.where(t < PA, q, P - 1), 0

    out = pl.pallas_call(
        _main_body,
        grid=(TOTAL,),
        in_specs=[lspec, lspec, lspec,
                  pl.BlockSpec((BLK, 3 * H), _ymap),
                  const((3, H)),
                  const((3, H, H)),
                  const((3, H)),
                  const((3, H, OUT)),
                  const((3, OUT))],
        out_specs=pl.BlockSpec((1, OUT), lambda t: (0, 0)),
        out_shape=jax.ShapeDtypeStruct((1, OUT), jnp.float32),
        scratch_shapes=[pltpu.VMEM((3, 1, N), jnp.float32),
                        pltpu.VMEM((3, 1, N), jnp.float32),
                        pltpu.VMEM((N, 3 * H), jnp.float32),
                        pltpu.VMEM((3, 2, BLK, BLK), jnp.float32),
                        pltpu.VMEM((3, 1, H), jnp.float32)],
    )(L0, L1, L2, y1, b1s, w2s, b2s, w3s, b3s)
    return out
